# baseline (speedup 1.0000x reference)
import functools

import jax
import jax.numpy as jnp
from jax import lax
from jax.experimental import pallas as pl
from jax.experimental.pallas import tpu as pltpu
from jax.experimental.pallas import tpu_sc as plsc

N = 10000
NPAD = 10240
DEG = 16
NB = 160000
NBP = 163840
DIM = 128
NW = 32


def _mesh():
    return plsc.VectorSubcoreMesh(
        core_axis_name="c", subcore_axis_name="s", num_cores=2, num_subcores=16
    )


def _wid():
    return lax.axis_index("s") * 2 + lax.axis_index("c")


def _linear(x, wt, b=None, act=None, R=640):
    M, K = x.shape
    Do = wt.shape[1]
    in_specs = [
        pl.BlockSpec((R, K), lambda i: (i, 0)),
        pl.BlockSpec((K, Do), lambda i: (0, 0)),
    ]
    args = [x, wt]
    if b is not None:
        in_specs.append(pl.BlockSpec((1, Do), lambda i: (0, 0)))
        args.append(b.reshape(1, Do))

    def body(*refs):
        x_ref, w_ref = refs[0], refs[1]
        o_ref = refs[-1]
        y = jnp.dot(x_ref[...], w_ref[...], preferred_element_type=jnp.float32)
        if b is not None:
            y = y + refs[2][...]
        if act == "relu":
            y = jnp.maximum(y, 0.0)
        o_ref[...] = y

    return pl.pallas_call(
        body,
        grid=(M // R,),
        in_specs=in_specs,
        out_specs=pl.BlockSpec((R, Do), lambda i: (i, 0)),
        out_shape=jax.ShapeDtypeStruct((M, Do), jnp.float32),
    )(*args)


def _adj_step(adjacency, hs, xs, BM=400):
    def body(a_ref, h_ref, x_ref, o_ref):
        o_ref[...] = x_ref[...] + jnp.dot(
            a_ref[...], h_ref[...], preferred_element_type=jnp.float32
        )

    return pl.pallas_call(
        body,
        grid=(N // BM,),
        in_specs=[
            pl.BlockSpec((BM, N), lambda i: (i, 0)),
            pl.BlockSpec((N, DIM), lambda i: (0, 0)),
            pl.BlockSpec((BM, DIM), lambda i: (i, 0)),
        ],
        out_specs=pl.BlockSpec((BM, DIM), lambda i: (i, 0)),
        out_shape=jax.ShapeDtypeStruct((N, DIM), jnp.float32),
    )(adjacency, hs, xs)


def _bf_init(bond_feature, wbond_t, b_bond, wb0_t):
    R = 1280
    nreal = NB // R

    def body(x_ref, w1_ref, b_ref, w2_ref, o1_ref, o2_ref):
        t = jnp.dot(x_ref[...], w1_ref[...],
                    preferred_element_type=jnp.float32) + b_ref[...]
        o1_ref[...] = t
        o2_ref[...] = jnp.dot(t, w2_ref[...],
                              preferred_element_type=jnp.float32)

    return pl.pallas_call(
        body,
        grid=(NBP // R,),
        in_specs=[
            pl.BlockSpec((R, 10), lambda i: (jnp.minimum(i, nreal - 1), 0)),
            pl.BlockSpec((10, DIM), lambda i: (0, 0)),
            pl.BlockSpec((1, DIM), lambda i: (0, 0)),
            pl.BlockSpec((DIM, DIM), lambda i: (0, 0)),
        ],
        out_specs=[
            pl.BlockSpec((R, DIM), lambda i: (i, 0)),
            pl.BlockSpec((R, DIM), lambda i: (i, 0)),
        ],
        out_shape=[
            jax.ShapeDtypeStruct((NBP, DIM), jnp.float32),
            jax.ShapeDtypeStruct((NBP, DIM), jnp.float32),
        ],
    )(bond_feature, wbond_t, b_bond.reshape(1, DIM), wb0_t)


def _epilogue(xs, af, words, wfc_t, bfc, wout_t, bout, wint_t, bint):
    def body(xs_ref, af_ref, w_ref, wfc_ref, bfc_ref, wout_ref, bout_ref,
             wint_ref, bint_ref, o_ref):
        s = jnp.sum(xs_ref[...] + af_ref[...], axis=0, keepdims=True) * (1.0 / N)
        m = jnp.dot(w_ref[...], wfc_ref[...], preferred_element_type=jnp.float32)
        m = m + bfc_ref[...]
        cat = jnp.concatenate([s, m], axis=1)
        for j in range(2):
            cat = jnp.dot(cat, wout_ref[j], preferred_element_type=jnp.float32)
            cat = jnp.maximum(cat + bout_ref[j], 0.0)
        o_ref[...] = (
            jnp.dot(cat, wint_ref[...], preferred_element_type=jnp.float32)
            + bint_ref[...]
        )

    return pl.pallas_call(
        body,
        out_shape=jax.ShapeDtypeStruct((1, 2), jnp.float32),
    )(xs, af, words.reshape(1, -1), wfc_t, bfc.reshape(1, -1), wout_t,
      bout.reshape(2, 1, 2 * DIM), wint_t, bint.reshape(1, -1))


def _sc_embed(table, idx):
    B = idx.shape[0]
    bpw = B // NW

    @functools.partial(
        pl.kernel,
        mesh=_mesh(),
        out_type=jax.ShapeDtypeStruct((B, DIM), jnp.float32),
        scratch_types=[
            pltpu.VMEM((bpw,), jnp.int32),
            pltpu.VMEM((bpw, DIM), jnp.float32),
            pltpu.SemaphoreType.DMA,
        ],
    )
    def k(table_h, idx_h, out_h, idx_v, rows_v, sem):
        base = _wid() * bpw
        pltpu.sync_copy(idx_h.at[pl.ds(base, bpw)], idx_v)
        cps = []
        for c in range(bpw // 64):
            sl = pl.ds(c * 64, 64)
            cps.append(pltpu.async_copy(table_h.at[idx_v.at[sl]],
                                        rows_v.at[sl, :], sem))
        for cp in cps:
            cp.wait()
        pltpu.sync_copy(rows_v, out_h.at[pl.ds(base, bpw), :])

    return k(table, idx)


def _sigmoid(x):
    return 1.0 / (1.0 + jnp.exp(-x))


K0 = 40
K1 = 40


def _wait_bytes(src_h, dst_ref, sem):
    pltpu.make_async_copy(src_h, dst_ref, sem).wait()


def _sc_nf(aw, bw, adl_flat, bdl_flat, af, bias):
    CA = 8
    R = CA * DEG
    KM = max(K0, K1)

    @functools.partial(
        pl.kernel,
        mesh=_mesh(),
        out_type=jax.ShapeDtypeStruct((NPAD, DIM), jnp.float32),
        scratch_types=[
            pltpu.VMEM((KM * R,), jnp.int32),
            pltpu.VMEM((KM * R,), jnp.int32),
            pltpu.VMEM((2, R, DIM), jnp.float32),
            pltpu.VMEM((2, R, DIM), jnp.float32),
            pltpu.VMEM((2, CA, DIM), jnp.float32),
            pltpu.VMEM((2, CA, DIM), jnp.float32),
            pltpu.VMEM((DIM,), jnp.float32),
            pltpu.SemaphoreType.DMA,
            pltpu.SemaphoreType.DMA,
            pltpu.SemaphoreType.DMA,
            pltpu.SemaphoreType.DMA,
        ],
    )
    def k(aw_h, bw_h, adl_h, bdl_h, af_h, bias_h, out_h,
          ia_v, ib_v, ar_v, br_v, af_v, oc_v, b_v, sema, semb, semf, sems):
        pltpu.sync_copy(bias_h, b_v)
        bias_vecs = [b_v[pl.ds(v * 16, 16)] for v in range(8)]

        def pipe(start, nk):
            pltpu.sync_copy(adl_h.at[pl.ds(start * R, nk * R)],
                            ia_v.at[pl.ds(0, nk * R)])
            pltpu.sync_copy(bdl_h.at[pl.ds(start * R, nk * R)],
                            ib_v.at[pl.ds(0, nk * R)])

            def issue(i, p):
                sl = pl.ds(i * R, R)
                rows = pl.ds((start + i) * CA, CA)
                pltpu.async_copy(aw_h.at[ia_v.at[sl]], ar_v.at[p], sema)
                pltpu.async_copy(bw_h.at[ib_v.at[sl]], br_v.at[p], semb)
                pltpu.async_copy(af_h.at[rows, :], af_v.at[p], semf)

            def half(j, p):
                i = 2 * j + p
                rows = pl.ds((start + i) * CA, CA)

                @pl.when(j > 0)
                def _():
                    _wait_bytes(af_h.at[rows, :], oc_v.at[p], sems)

                _wait_bytes(af_h.at[pl.ds(0, R), :], ar_v.at[p], sema)
                _wait_bytes(af_h.at[pl.ds(0, R), :], br_v.at[p], semb)
                _wait_bytes(af_h.at[pl.ds(0, CA), :], af_v.at[p], semf)

                def atom(a, _):
                    r0 = a * DEG

                    def dbody(d2, accs):
                        out = list(accs)
                        for u in range(2):
                            r = r0 + d2 * 2 + u
                            for v in range(8):
                                sl = pl.ds(v * 16, 16)
                                x = (ar_v[p, r, sl] + br_v[p, r, sl]
                                     + bias_vecs[v])
                                out[v] = (out[v] + jnp.maximum(x, 0.0)
                                          + 0.01 * jnp.minimum(x, 0.0))
                        return tuple(out)

                    accs = lax.fori_loop(
                        0, DEG // 2, dbody,
                        tuple(jnp.zeros((16,), jnp.float32) for _ in range(8)))
                    for v in range(8):
                        sl = pl.ds(v * 16, 16)
                        oc_v[p, a, sl] = _sigmoid(af_v[p, a, sl] + accs[v])
                    return 0

                lax.fori_loop(0, CA, atom, 0)
                pltpu.async_copy(oc_v.at[p], out_h.at[rows, :], sems)

                @pl.when(i + 2 < nk)
                def _():
                    issue(i + 2, p)

            issue(0, 0)
            issue(1, 1)

            def body(j, _):
                half(j, 0)
                half(j, 1)
                return 0

            lax.fori_loop(0, nk // 2, body, 0)
            _wait_bytes(af_h.at[pl.ds(0, CA), :], oc_v.at[0], sems)
            _wait_bytes(af_h.at[pl.ds(0, CA), :], oc_v.at[1], sems)

        c_ax = lax.axis_index("c")
        s_ax = lax.axis_index("s")

        @pl.when(c_ax == 0)
        def _():
            pipe(s_ax * K0, K0)

        @pl.when(c_ax == 1)
        def _():
            pipe(16 * K0 + s_ax * K1, K1)

    return k(aw, bw, adl_flat, bdl_flat, af, bias)


def _sc_side(sw, i0, i1):
    E = 128
    KM = max(K0, K1)

    @functools.partial(
        pl.kernel,
        mesh=_mesh(),
        out_type=jax.ShapeDtypeStruct((NBP, DIM), jnp.float32),
        scratch_types=[
            pltpu.VMEM((KM * E,), jnp.int32),
            pltpu.VMEM((KM * E,), jnp.int32),
            pltpu.VMEM((2, E, DIM), jnp.float32),
            pltpu.VMEM((2, E, DIM), jnp.float32),
            pltpu.VMEM((2, E, DIM), jnp.float32),
            pltpu.SemaphoreType.DMA,
            pltpu.SemaphoreType.DMA,
            pltpu.SemaphoreType.DMA,
        ],
    )
    def k(sw_h, i0_h, i1_h, out_h, i0_v, i1_v, g0_v, g1_v, oc_v,
          sem0, sem1, sems):
        sw_s = sw_h

        def pipe(start, nk):
            pltpu.sync_copy(i0_h.at[pl.ds(start * E, nk * E)],
                            i0_v.at[pl.ds(0, nk * E)])
            pltpu.sync_copy(i1_h.at[pl.ds(start * E, nk * E)],
                            i1_v.at[pl.ds(0, nk * E)])

            def issue(i, p):
                sl = pl.ds(i * E, E)
                pltpu.async_copy(sw_s.at[i0_v.at[sl]], g0_v.at[p], sem0)
                pltpu.async_copy(sw_s.at[i1_v.at[sl]], g1_v.at[p], sem1)

            def half(j, p):
                i = 2 * j + p
                rows = pl.ds((start + i) * E, E)

                @pl.when(j > 0)
                def _():
                    _wait_bytes(sw_h.at[pl.ds(0, E), :], oc_v.at[p], sems)

                _wait_bytes(sw_h.at[pl.ds(0, E), :], g0_v.at[p], sem0)
                _wait_bytes(sw_h.at[pl.ds(0, E), :], g1_v.at[p], sem1)

                def row(e2, _):
                    for u in range(2):
                        e = e2 * 2 + u
                        for v in range(8):
                            sl = pl.ds(v * 16, 16)
                            oc_v[p, e, sl] = g0_v[p, e, sl] + g1_v[p, e, sl]
                    return 0

                lax.fori_loop(0, E // 2, row, 0)
                pltpu.async_copy(oc_v.at[p], out_h.at[rows, :], sems)

                @pl.when(i + 2 < nk)
                def _():
                    issue(i + 2, p)

            issue(0, 0)
            issue(1, 1)

            def body(j, _):
                half(j, 0)
                half(j, 1)
                return 0

            lax.fori_loop(0, nk // 2, body, 0)
            _wait_bytes(sw_h.at[pl.ds(0, E), :], oc_v.at[0], sems)
            _wait_bytes(sw_h.at[pl.ds(0, E), :], oc_v.at[1], sems)

        c_ax = lax.axis_index("c")
        s_ax = lax.axis_index("s")

        @pl.when(c_ax == 0)
        def _():
            pipe(s_ax * K0, K0)

        @pl.when(c_ax == 1)
        def _():
            pipe(16 * K0 + s_ax * K1, K1)

    return k(sw, i0, i1)


def _bond_fuse(bf, side, b, wt_next, need_bf, R=2048):
    out_shapes = [jax.ShapeDtypeStruct((NBP, DIM), jnp.float32)]
    out_specs = [pl.BlockSpec((R, DIM), lambda i: (i, 0))]
    if need_bf:
        out_shapes.append(jax.ShapeDtypeStruct((NBP, DIM), jnp.float32))
        out_specs.append(pl.BlockSpec((R, DIM), lambda i: (i, 0)))

    def body(bf_ref, sd_ref, b_ref, w_ref, o_ref, *rest):
        s = _sigmoid(bf_ref[...] + sd_ref[...] + b_ref[...])
        o_ref[...] = jnp.dot(s, w_ref[...], preferred_element_type=jnp.float32)
        if need_bf:
            rest[0][...] = s

    res = pl.pallas_call(
        body,
        grid=(NBP // R,),
        in_specs=[
            pl.BlockSpec((R, DIM), lambda i: (i, 0)),
            pl.BlockSpec((R, DIM), lambda i: (i, 0)),
            pl.BlockSpec((1, DIM), lambda i: (0, 0)),
            pl.BlockSpec((DIM, DIM), lambda i: (0, 0)),
        ],
        out_specs=out_specs,
        out_shape=out_shapes,
    )(bf, side, b.reshape(1, DIM), wt_next)
    return res if need_bf else (res[0], None)


def kernel(fingerprints, atom_degree_list, bond_feature, bond_degree_list,
           i_bond_j, adjacency, words, embed_table, W_bond, b_bond, W_nfc,
           b_nfc, W_sfc, b_sfc, W_sub, b_sub, W_fc, b_fc, W_out, b_out,
           W_int, b_int):
    def _padi(a, total, mod):
        pad = jnp.arange(total - a.shape[0], dtype=jnp.int32) % mod
        return jnp.concatenate([a.astype(jnp.int32), pad])

    fp = _padi(fingerprints, NPAD, 100000)
    xs = _sc_embed(embed_table, fp)[:N]

    for i in range(2):
        hs = _linear(xs, W_sub[i].T, b_sub[i], "relu", R=1000)
        xs = _adj_step(adjacency, hs, xs)

    bf, bw = _bf_init(bond_feature, W_bond.T, b_bond, W_nfc[0, :, DIM:].T)
    af_p = jnp.pad(xs, ((0, NPAD - N), (0, 0)))
    adlf = _padi(atom_degree_list.reshape(-1), NPAD * DEG, N)
    bdlf = _padi(bond_degree_list.reshape(-1), NPAD * DEG, NB)
    i0 = _padi(i_bond_j[:, 0], NBP, N)
    i1 = _padi(i_bond_j[:, 1], NBP, N)

    for i in range(3):
        aw = _linear(af_p, W_nfc[i, :, :DIM].T, None, None, R=1024)
        af_p = _sc_nf(aw, bw, adlf, bdlf, af_p, b_nfc[i])
        if i < 2:
            sw = _linear(af_p, W_sfc[i].T, None, None, R=1024)
            side = _sc_side(sw, i0, i1)
            bw, bf_new = _bond_fuse(bf, side, b_sfc[i],
                                    W_nfc[i + 1, :, DIM:].T, need_bf=(i == 0))
            if i == 0:
                bf = bf_new

    return _epilogue(xs, af_p[:N], words, W_fc.T, b_fc,
                     jnp.transpose(W_out, (0, 2, 1)), b_out, W_int.T, b_int)


# speedup vs baseline: 1.0554x; 1.0554x over previous
import functools

import jax
import jax.numpy as jnp
from jax import lax
from jax.experimental import pallas as pl
from jax.experimental.pallas import tpu as pltpu
from jax.experimental.pallas import tpu_sc as plsc

N = 10000
NPAD = 10240
DEG = 16
NB = 160000
NBP = 163840
DIM = 128
NW = 32


def _mesh():
    return plsc.VectorSubcoreMesh(
        core_axis_name="c", subcore_axis_name="s", num_cores=2, num_subcores=16
    )


def _wid():
    return lax.axis_index("s") * 2 + lax.axis_index("c")


def _linear(x, wt, b=None, act=None, R=640):
    M, K = x.shape
    Do = wt.shape[1]
    in_specs = [
        pl.BlockSpec((R, K), lambda i: (i, 0)),
        pl.BlockSpec((K, Do), lambda i: (0, 0)),
    ]
    args = [x, wt]
    if b is not None:
        in_specs.append(pl.BlockSpec((1, Do), lambda i: (0, 0)))
        args.append(b.reshape(1, Do))

    def body(*refs):
        x_ref, w_ref = refs[0], refs[1]
        o_ref = refs[-1]
        y = jnp.dot(x_ref[...], w_ref[...], preferred_element_type=jnp.float32)
        if b is not None:
            y = y + refs[2][...]
        if act == "relu":
            y = jnp.maximum(y, 0.0)
        o_ref[...] = y

    return pl.pallas_call(
        body,
        grid=(M // R,),
        in_specs=in_specs,
        out_specs=pl.BlockSpec((R, Do), lambda i: (i, 0)),
        out_shape=jax.ShapeDtypeStruct((M, Do), jnp.float32),
    )(*args)


def _adj_step(adjacency, hs, xs, BM=400):
    def body(a_ref, h_ref, x_ref, o_ref):
        o_ref[...] = x_ref[...] + jnp.dot(
            a_ref[...], h_ref[...], preferred_element_type=jnp.float32
        )

    return pl.pallas_call(
        body,
        grid=(N // BM,),
        in_specs=[
            pl.BlockSpec((BM, N), lambda i: (i, 0)),
            pl.BlockSpec((N, DIM), lambda i: (0, 0)),
            pl.BlockSpec((BM, DIM), lambda i: (i, 0)),
        ],
        out_specs=pl.BlockSpec((BM, DIM), lambda i: (i, 0)),
        out_shape=jax.ShapeDtypeStruct((N, DIM), jnp.float32),
    )(adjacency, hs, xs)


def _bf_init(bond_feature, wbond_t, b_bond, wb0_t):
    R = 1280
    nreal = NB // R

    def body(x_ref, w1_ref, b_ref, w2_ref, o1_ref, o2_ref):
        t = jnp.dot(x_ref[...], w1_ref[...],
                    preferred_element_type=jnp.float32) + b_ref[...]
        o1_ref[...] = t
        o2_ref[...] = jnp.dot(t, w2_ref[...],
                              preferred_element_type=jnp.float32)

    return pl.pallas_call(
        body,
        grid=(NBP // R,),
        in_specs=[
            pl.BlockSpec((R, 10), lambda i: (jnp.minimum(i, nreal - 1), 0)),
            pl.BlockSpec((10, DIM), lambda i: (0, 0)),
            pl.BlockSpec((1, DIM), lambda i: (0, 0)),
            pl.BlockSpec((DIM, DIM), lambda i: (0, 0)),
        ],
        out_specs=[
            pl.BlockSpec((R, DIM), lambda i: (i, 0)),
            pl.BlockSpec((R, DIM), lambda i: (i, 0)),
        ],
        out_shape=[
            jax.ShapeDtypeStruct((NBP, DIM), jnp.float32),
            jax.ShapeDtypeStruct((NBP, DIM), jnp.float32),
        ],
    )(bond_feature, wbond_t, b_bond.reshape(1, DIM), wb0_t)


def _epilogue(xs, af, words, wfc_t, bfc, wout_t, bout, wint_t, bint):
    def body(xs_ref, af_ref, w_ref, wfc_ref, bfc_ref, wout_ref, bout_ref,
             wint_ref, bint_ref, o_ref):
        s = jnp.sum(xs_ref[...] + af_ref[...], axis=0, keepdims=True) * (1.0 / N)
        m = jnp.dot(w_ref[...], wfc_ref[...], preferred_element_type=jnp.float32)
        m = m + bfc_ref[...]
        cat = jnp.concatenate([s, m], axis=1)
        for j in range(2):
            cat = jnp.dot(cat, wout_ref[j], preferred_element_type=jnp.float32)
            cat = jnp.maximum(cat + bout_ref[j], 0.0)
        o_ref[...] = (
            jnp.dot(cat, wint_ref[...], preferred_element_type=jnp.float32)
            + bint_ref[...]
        )

    return pl.pallas_call(
        body,
        out_shape=jax.ShapeDtypeStruct((1, 2), jnp.float32),
    )(xs, af, words.reshape(1, -1), wfc_t, bfc.reshape(1, -1), wout_t,
      bout.reshape(2, 1, 2 * DIM), wint_t, bint.reshape(1, -1))


def _sc_embed(table, idx):
    B = idx.shape[0]
    bpw = B // NW

    @functools.partial(
        pl.kernel,
        mesh=_mesh(),
        out_type=jax.ShapeDtypeStruct((B, DIM), jnp.float32),
        scratch_types=[
            pltpu.VMEM((bpw,), jnp.int32),
            pltpu.VMEM((bpw, DIM), jnp.float32),
            pltpu.SemaphoreType.DMA,
        ],
    )
    def k(table_h, idx_h, out_h, idx_v, rows_v, sem):
        base = _wid() * bpw
        pltpu.sync_copy(idx_h.at[pl.ds(base, bpw)], idx_v)
        cps = []
        for c in range(bpw // 64):
            sl = pl.ds(c * 64, 64)
            cps.append(pltpu.async_copy(table_h.at[idx_v.at[sl]],
                                        rows_v.at[sl, :], sem))
        for cp in cps:
            cp.wait()
        pltpu.sync_copy(rows_v, out_h.at[pl.ds(base, bpw), :])

    return k(table, idx)


def _sigmoid(x):
    return 1.0 / (1.0 + jnp.exp(-x))


K0 = 40
K1 = 40


def _wait_bytes(src_h, dst_ref, sem):
    pltpu.make_async_copy(src_h, dst_ref, sem).wait()


def _sc_nf(aw, bw, adl_flat, bdl_flat, af, bias):
    CA = 8
    R = CA * DEG
    KM = max(K0, K1)

    @functools.partial(
        pl.kernel,
        mesh=_mesh(),
        out_type=jax.ShapeDtypeStruct((NPAD, DIM), jnp.float32),
        scratch_types=[
            pltpu.VMEM((KM * R,), jnp.int32),
            pltpu.VMEM((KM * R,), jnp.int32),
            pltpu.VMEM((2, R, DIM), jnp.float32),
            pltpu.VMEM((2, R, DIM), jnp.float32),
            pltpu.VMEM((2, CA, DIM), jnp.float32),
            pltpu.VMEM((2, CA, DIM), jnp.float32),
            pltpu.VMEM((DIM,), jnp.float32),
            pltpu.SemaphoreType.DMA,
            pltpu.SemaphoreType.DMA,
            pltpu.SemaphoreType.DMA,
            pltpu.SemaphoreType.DMA,
        ],
    )
    def k(aw_h, bw_h, adl_h, bdl_h, af_h, bias_h, out_h,
          ia_v, ib_v, ar_v, br_v, af_v, oc_v, b_v, sema, semb, semf, sems):
        pltpu.sync_copy(bias_h, b_v)
        bias_vecs = [b_v[pl.ds(v * 16, 16)] for v in range(8)]

        def pipe(start, nk):
            pltpu.sync_copy(adl_h.at[pl.ds(start * R, nk * R)],
                            ia_v.at[pl.ds(0, nk * R)])
            pltpu.sync_copy(bdl_h.at[pl.ds(start * R, nk * R)],
                            ib_v.at[pl.ds(0, nk * R)])

            def issue(i, p):
                sl = pl.ds(i * R, R)
                rows = pl.ds((start + i) * CA, CA)
                pltpu.async_copy(aw_h.at[ia_v.at[sl]], ar_v.at[p], sema)
                pltpu.async_copy(bw_h.at[ib_v.at[sl]], br_v.at[p], semb)
                pltpu.async_copy(af_h.at[rows, :], af_v.at[p], semf)

            def half(j, p):
                i = 2 * j + p
                rows = pl.ds((start + i) * CA, CA)

                @pl.when(j > 0)
                def _():
                    _wait_bytes(af_h.at[rows, :], oc_v.at[p], sems)

                _wait_bytes(af_h.at[pl.ds(0, R), :], ar_v.at[p], sema)
                _wait_bytes(af_h.at[pl.ds(0, R), :], br_v.at[p], semb)
                _wait_bytes(af_h.at[pl.ds(0, CA), :], af_v.at[p], semf)

                def atom(a, _):
                    r0 = a * DEG

                    def dbody(d, accs):
                        r = r0 + d
                        out = []
                        for v in range(8):
                            sl = pl.ds(v * 16, 16)
                            x = ar_v[p, r, sl] + br_v[p, r, sl] + bias_vecs[v]
                            out.append(accs[v] + jnp.maximum(x, 0.0)
                                       + 0.01 * jnp.minimum(x, 0.0))
                        return tuple(out)

                    accs = lax.fori_loop(
                        0, DEG, dbody,
                        tuple(jnp.zeros((16,), jnp.float32) for _ in range(8)))
                    for v in range(8):
                        sl = pl.ds(v * 16, 16)
                        oc_v[p, a, sl] = _sigmoid(af_v[p, a, sl] + accs[v])
                    return 0

                lax.fori_loop(0, CA, atom, 0)
                pltpu.async_copy(oc_v.at[p], out_h.at[rows, :], sems)

                @pl.when(i + 2 < nk)
                def _():
                    issue(i + 2, p)

            issue(0, 0)
            issue(1, 1)

            def body(j, _):
                half(j, 0)
                half(j, 1)
                return 0

            lax.fori_loop(0, nk // 2, body, 0)
            _wait_bytes(af_h.at[pl.ds(0, CA), :], oc_v.at[0], sems)
            _wait_bytes(af_h.at[pl.ds(0, CA), :], oc_v.at[1], sems)

        c_ax = lax.axis_index("c")
        s_ax = lax.axis_index("s")

        @pl.when(c_ax == 0)
        def _():
            pipe(s_ax * K0, K0)

        @pl.when(c_ax == 1)
        def _():
            pipe(16 * K0 + s_ax * K1, K1)

    return k(aw, bw, adl_flat, bdl_flat, af, bias)


def _sc_side(sw, i0, i1):
    E = 128
    KM = max(K0, K1)

    @functools.partial(
        pl.kernel,
        mesh=_mesh(),
        out_type=jax.ShapeDtypeStruct((NBP, DIM), jnp.float32),
        scratch_types=[
            pltpu.VMEM((KM * E,), jnp.int32),
            pltpu.VMEM((KM * E,), jnp.int32),
            pltpu.VMEM((2, E, DIM), jnp.float32),
            pltpu.VMEM((2, E, DIM), jnp.float32),
            pltpu.VMEM((2, E, DIM), jnp.float32),
            pltpu.SemaphoreType.DMA,
            pltpu.SemaphoreType.DMA,
            pltpu.SemaphoreType.DMA,
        ],
    )
    def k(sw_h, i0_h, i1_h, out_h, i0_v, i1_v, g0_v, g1_v, oc_v,
          sem0, sem1, sems):
        sw_s = sw_h

        def pipe(start, nk):
            pltpu.sync_copy(i0_h.at[pl.ds(start * E, nk * E)],
                            i0_v.at[pl.ds(0, nk * E)])
            pltpu.sync_copy(i1_h.at[pl.ds(start * E, nk * E)],
                            i1_v.at[pl.ds(0, nk * E)])

            def issue(i, p):
                sl = pl.ds(i * E, E)
                pltpu.async_copy(sw_s.at[i0_v.at[sl]], g0_v.at[p], sem0)
                pltpu.async_copy(sw_s.at[i1_v.at[sl]], g1_v.at[p], sem1)

            def half(j, p):
                i = 2 * j + p
                rows = pl.ds((start + i) * E, E)

                @pl.when(j > 0)
                def _():
                    _wait_bytes(sw_h.at[pl.ds(0, E), :], oc_v.at[p], sems)

                _wait_bytes(sw_h.at[pl.ds(0, E), :], g0_v.at[p], sem0)
                _wait_bytes(sw_h.at[pl.ds(0, E), :], g1_v.at[p], sem1)

                def row(e, _):
                    for v in range(8):
                        sl = pl.ds(v * 16, 16)
                        oc_v[p, e, sl] = g0_v[p, e, sl] + g1_v[p, e, sl]
                    return 0

                lax.fori_loop(0, E, row, 0)
                pltpu.async_copy(oc_v.at[p], out_h.at[rows, :], sems)

                @pl.when(i + 2 < nk)
                def _():
                    issue(i + 2, p)

            issue(0, 0)
            issue(1, 1)

            def body(j, _):
                half(j, 0)
                half(j, 1)
                return 0

            lax.fori_loop(0, nk // 2, body, 0)
            _wait_bytes(sw_h.at[pl.ds(0, E), :], oc_v.at[0], sems)
            _wait_bytes(sw_h.at[pl.ds(0, E), :], oc_v.at[1], sems)

        c_ax = lax.axis_index("c")
        s_ax = lax.axis_index("s")

        @pl.when(c_ax == 0)
        def _():
            pipe(s_ax * K0, K0)

        @pl.when(c_ax == 1)
        def _():
            pipe(16 * K0 + s_ax * K1, K1)

    return k(sw, i0, i1)


def _bond_fuse(bf, side, b, wt_next, need_bf, R=2048):
    out_shapes = [jax.ShapeDtypeStruct((NBP, DIM), jnp.float32)]
    out_specs = [pl.BlockSpec((R, DIM), lambda i: (i, 0))]
    if need_bf:
        out_shapes.append(jax.ShapeDtypeStruct((NBP, DIM), jnp.float32))
        out_specs.append(pl.BlockSpec((R, DIM), lambda i: (i, 0)))

    def body(bf_ref, sd_ref, b_ref, w_ref, o_ref, *rest):
        s = _sigmoid(bf_ref[...] + sd_ref[...] + b_ref[...])
        o_ref[...] = jnp.dot(s, w_ref[...], preferred_element_type=jnp.float32)
        if need_bf:
            rest[0][...] = s

    res = pl.pallas_call(
        body,
        grid=(NBP // R,),
        in_specs=[
            pl.BlockSpec((R, DIM), lambda i: (i, 0)),
            pl.BlockSpec((R, DIM), lambda i: (i, 0)),
            pl.BlockSpec((1, DIM), lambda i: (0, 0)),
            pl.BlockSpec((DIM, DIM), lambda i: (0, 0)),
        ],
        out_specs=out_specs,
        out_shape=out_shapes,
    )(bf, side, b.reshape(1, DIM), wt_next)
    return res if need_bf else (res[0], None)


def kernel(fingerprints, atom_degree_list, bond_feature, bond_degree_list,
           i_bond_j, adjacency, words, embed_table, W_bond, b_bond, W_nfc,
           b_nfc, W_sfc, b_sfc, W_sub, b_sub, W_fc, b_fc, W_out, b_out,
           W_int, b_int):
    def _padi(a, total, mod):
        pad = jnp.arange(total - a.shape[0], dtype=jnp.int32) % mod
        return jnp.concatenate([a.astype(jnp.int32), pad])

    fp = _padi(fingerprints, NPAD, 100000)
    xs = _sc_embed(embed_table, fp)[:N]

    for i in range(2):
        hs = _linear(xs, W_sub[i].T, b_sub[i], "relu", R=1000)
        xs = _adj_step(adjacency, hs, xs)

    bf, bw = _bf_init(bond_feature, W_bond.T, b_bond, W_nfc[0, :, DIM:].T)
    af_p = jnp.pad(xs, ((0, NPAD - N), (0, 0)))
    adlf = _padi(atom_degree_list.reshape(-1), NPAD * DEG, N)
    bdlf = _padi(bond_degree_list.reshape(-1), NPAD * DEG, NB)
    i0 = _padi(i_bond_j[:, 0], NBP, N)
    i1 = _padi(i_bond_j[:, 1], NBP, N)

    for i in range(3):
        aw = _linear(af_p, W_nfc[i, :, :DIM].T, None, None, R=1024)
        af_p = _sc_nf(aw, bw, adlf, bdlf, af_p, b_nfc[i])
        if i < 2:
            sw = _linear(af_p, W_sfc[i].T, None, None, R=1024)
            side = _sc_side(sw, i0, i1)
            bw, bf_new = _bond_fuse(bf, side, b_sfc[i],
                                    W_nfc[i + 1, :, DIM:].T, need_bf=(i == 0))
            if i == 0:
                bf = bf_new

    return _epilogue(xs, af_p[:N], words, W_fc.T, b_fc,
                     jnp.transpose(W_out, (0, 2, 1)), b_out, W_int.T, b_int)


# speedup vs baseline: 1.0871x; 1.0300x over previous
import functools

import jax
import jax.numpy as jnp
from jax import lax
from jax.experimental import pallas as pl
from jax.experimental.pallas import tpu as pltpu
from jax.experimental.pallas import tpu_sc as plsc

N = 10000
NPAD = 10240
DEG = 16
NB = 160000
NBP = 163840
DIM = 128
NW = 32


def _mesh():
    return plsc.VectorSubcoreMesh(
        core_axis_name="c", subcore_axis_name="s", num_cores=2, num_subcores=16
    )


def _wid():
    return lax.axis_index("s") * 2 + lax.axis_index("c")


def _linear(x, wt, b=None, act=None, R=640):
    M, K = x.shape
    Do = wt.shape[1]
    in_specs = [
        pl.BlockSpec((R, K), lambda i: (i, 0)),
        pl.BlockSpec((K, Do), lambda i: (0, 0)),
    ]
    args = [x, wt]
    if b is not None:
        in_specs.append(pl.BlockSpec((1, Do), lambda i: (0, 0)))
        args.append(b.reshape(1, Do))

    def body(*refs):
        x_ref, w_ref = refs[0], refs[1]
        o_ref = refs[-1]
        y = jnp.dot(x_ref[...], w_ref[...], preferred_element_type=jnp.float32)
        if b is not None:
            y = y + refs[2][...]
        if act == "relu":
            y = jnp.maximum(y, 0.0)
        o_ref[...] = y

    return pl.pallas_call(
        body,
        grid=(M // R,),
        in_specs=in_specs,
        out_specs=pl.BlockSpec((R, Do), lambda i: (i, 0)),
        out_shape=jax.ShapeDtypeStruct((M, Do), jnp.float32),
    )(*args)


def _adj_step(adjacency, hs, xs, BM=400):
    def body(a_ref, h_ref, x_ref, o_ref):
        o_ref[...] = x_ref[...] + jnp.dot(
            a_ref[...], h_ref[...], preferred_element_type=jnp.float32
        )

    return pl.pallas_call(
        body,
        grid=(N // BM,),
        in_specs=[
            pl.BlockSpec((BM, N), lambda i: (i, 0)),
            pl.BlockSpec((N, DIM), lambda i: (0, 0)),
            pl.BlockSpec((BM, DIM), lambda i: (i, 0)),
        ],
        out_specs=pl.BlockSpec((BM, DIM), lambda i: (i, 0)),
        out_shape=jax.ShapeDtypeStruct((N, DIM), jnp.float32),
    )(adjacency, hs, xs)


def _bf_init(bond_feature, wbond_t, b_bond, wb0_t):
    R = 1280
    nreal = NB // R

    def body(x_ref, w1_ref, b_ref, w2_ref, o1_ref, o2_ref):
        t = jnp.dot(x_ref[...], w1_ref[...],
                    preferred_element_type=jnp.float32) + b_ref[...]
        o1_ref[...] = t
        o2_ref[...] = jnp.dot(t, w2_ref[...],
                              preferred_element_type=jnp.float32)

    return pl.pallas_call(
        body,
        grid=(NBP // R,),
        in_specs=[
            pl.BlockSpec((R, 10), lambda i: (jnp.minimum(i, nreal - 1), 0)),
            pl.BlockSpec((10, DIM), lambda i: (0, 0)),
            pl.BlockSpec((1, DIM), lambda i: (0, 0)),
            pl.BlockSpec((DIM, DIM), lambda i: (0, 0)),
        ],
        out_specs=[
            pl.BlockSpec((R, DIM), lambda i: (i, 0)),
            pl.BlockSpec((R, DIM), lambda i: (i, 0)),
        ],
        out_shape=[
            jax.ShapeDtypeStruct((NBP, DIM), jnp.float32),
            jax.ShapeDtypeStruct((NBP, DIM), jnp.float32),
        ],
    )(bond_feature, wbond_t, b_bond.reshape(1, DIM), wb0_t)


def _epilogue(xs, af, words, wfc_t, bfc, wout_t, bout, wint_t, bint):
    def body(xs_ref, af_ref, w_ref, wfc_ref, bfc_ref, wout_ref, bout_ref,
             wint_ref, bint_ref, o_ref):
        s = jnp.sum(xs_ref[...] + af_ref[...], axis=0, keepdims=True) * (1.0 / N)
        m = jnp.dot(w_ref[...], wfc_ref[...], preferred_element_type=jnp.float32)
        m = m + bfc_ref[...]
        cat = jnp.concatenate([s, m], axis=1)
        for j in range(2):
            cat = jnp.dot(cat, wout_ref[j], preferred_element_type=jnp.float32)
            cat = jnp.maximum(cat + bout_ref[j], 0.0)
        o_ref[...] = (
            jnp.dot(cat, wint_ref[...], preferred_element_type=jnp.float32)
            + bint_ref[...]
        )

    return pl.pallas_call(
        body,
        out_shape=jax.ShapeDtypeStruct((1, 2), jnp.float32),
    )(xs, af, words.reshape(1, -1), wfc_t, bfc.reshape(1, -1), wout_t,
      bout.reshape(2, 1, 2 * DIM), wint_t, bint.reshape(1, -1))


def _sc_embed(table, idx):
    B = idx.shape[0]
    bpw = B // NW

    @functools.partial(
        pl.kernel,
        mesh=_mesh(),
        out_type=jax.ShapeDtypeStruct((B, DIM), jnp.float32),
        scratch_types=[
            pltpu.VMEM((bpw,), jnp.int32),
            pltpu.VMEM((bpw, DIM), jnp.float32),
            pltpu.SemaphoreType.DMA,
        ],
    )
    def k(table_h, idx_h, out_h, idx_v, rows_v, sem):
        base = _wid() * bpw
        pltpu.sync_copy(idx_h.at[pl.ds(base, bpw)], idx_v)
        cps = []
        for c in range(bpw // 64):
            sl = pl.ds(c * 64, 64)
            cps.append(pltpu.async_copy(table_h.at[idx_v.at[sl]],
                                        rows_v.at[sl, :], sem))
        for cp in cps:
            cp.wait()
        pltpu.sync_copy(rows_v, out_h.at[pl.ds(base, bpw), :])

    return k(table, idx)


def _sigmoid(x):
    return 1.0 / (1.0 + jnp.exp(-x))


K0 = 40
K1 = 40


def _wait_bytes(src_h, dst_ref, sem):
    pltpu.make_async_copy(src_h, dst_ref, sem).wait()


def _sc_nf(aw, bw, adl_flat, bdl_flat, af, bias):
    CA = 8
    R = CA * DEG
    KM = max(K0, K1)

    @functools.partial(
        pl.kernel,
        mesh=_mesh(),
        out_type=jax.ShapeDtypeStruct((NPAD, DIM), jnp.float32),
        scratch_types=[
            pltpu.VMEM((KM * R,), jnp.int32),
            pltpu.VMEM((KM * R,), jnp.int32),
            pltpu.VMEM((2, R, DIM), jnp.float32),
            pltpu.VMEM((2, R, DIM), jnp.float32),
            pltpu.VMEM((2, CA, DIM), jnp.float32),
            pltpu.VMEM((2, CA, DIM), jnp.float32),
            pltpu.VMEM((DIM,), jnp.float32),
            pltpu.SemaphoreType.DMA,
            pltpu.SemaphoreType.DMA,
            pltpu.SemaphoreType.DMA,
            pltpu.SemaphoreType.DMA,
        ],
    )
    def k(aw_h, bw_h, adl_h, bdl_h, af_h, bias_h, out_h,
          ia_v, ib_v, ar_v, br_v, af_v, oc_v, b_v, sema, semb, semf, sems):
        pltpu.sync_copy(bias_h, b_v)
        bias_vecs = [b_v[pl.ds(v * 16, 16)] for v in range(8)]

        def pipe(start, nk):
            pltpu.sync_copy(adl_h.at[pl.ds(start * R, nk * R)],
                            ia_v.at[pl.ds(0, nk * R)])
            pltpu.sync_copy(bdl_h.at[pl.ds(start * R, nk * R)],
                            ib_v.at[pl.ds(0, nk * R)])

            def issue(i, p):
                sl = pl.ds(i * R, R)
                rows = pl.ds((start + i) * CA, CA)
                pltpu.async_copy(aw_h.at[ia_v.at[sl]], ar_v.at[p], sema)
                pltpu.async_copy(bw_h.at[ib_v.at[sl]], br_v.at[p], semb)
                pltpu.async_copy(af_h.at[rows, :], af_v.at[p], semf)

            def half(j, p):
                i = 2 * j + p
                rows = pl.ds((start + i) * CA, CA)

                @pl.when(j > 0)
                def _():
                    _wait_bytes(af_h.at[rows, :], oc_v.at[p], sems)

                _wait_bytes(af_h.at[pl.ds(0, R), :], ar_v.at[p], sema)
                _wait_bytes(af_h.at[pl.ds(0, R), :], br_v.at[p], semb)
                _wait_bytes(af_h.at[pl.ds(0, CA), :], af_v.at[p], semf)

                def atom(a, _):
                    r0 = a * DEG

                    def dbody(d, accs):
                        r = r0 + d
                        out = []
                        for v in range(8):
                            sl = pl.ds(v * 16, 16)
                            x = ar_v[p, r, sl] + br_v[p, r, sl] + bias_vecs[v]
                            out.append(accs[v] + jnp.maximum(x, 0.0)
                                       + 0.01 * jnp.minimum(x, 0.0))
                        return tuple(out)

                    accs = lax.fori_loop(
                        0, DEG, dbody,
                        tuple(jnp.zeros((16,), jnp.float32) for _ in range(8)))
                    for v in range(8):
                        sl = pl.ds(v * 16, 16)
                        oc_v[p, a, sl] = _sigmoid(af_v[p, a, sl] + accs[v])
                    return 0

                lax.fori_loop(0, CA, atom, 0)
                pltpu.async_copy(oc_v.at[p], out_h.at[rows, :], sems)

                @pl.when(i + 2 < nk)
                def _():
                    issue(i + 2, p)

            issue(0, 0)
            issue(1, 1)

            def body(j, _):
                half(j, 0)
                half(j, 1)
                return 0

            lax.fori_loop(0, nk // 2, body, 0)
            _wait_bytes(af_h.at[pl.ds(0, CA), :], oc_v.at[0], sems)
            _wait_bytes(af_h.at[pl.ds(0, CA), :], oc_v.at[1], sems)

        c_ax = lax.axis_index("c")
        s_ax = lax.axis_index("s")

        @pl.when(c_ax == 0)
        def _():
            pipe(s_ax * K0, K0)

        @pl.when(c_ax == 1)
        def _():
            pipe(16 * K0 + s_ax * K1, K1)

    return k(aw, bw, adl_flat, bdl_flat, af, bias)


def _sc_side(sw, i0, i1):
    E = 128
    KM = max(K0, K1)

    @functools.partial(
        pl.kernel,
        mesh=_mesh(),
        out_type=jax.ShapeDtypeStruct((NBP, DIM), jnp.float32),
        scratch_types=[
            pltpu.VMEM((KM * E,), jnp.int32),
            pltpu.VMEM((KM * E,), jnp.int32),
            pltpu.VMEM((2, E, DIM), jnp.float32),
            pltpu.VMEM((2, E, DIM), jnp.float32),
            pltpu.VMEM((2, E, DIM), jnp.float32),
            pltpu.SemaphoreType.DMA,
            pltpu.SemaphoreType.DMA,
            pltpu.SemaphoreType.DMA,
        ],
    )
    def k(sw_h, i0_h, i1_h, out_h, i0_v, i1_v, g0_v, g1_v, oc_v,
          sem0, sem1, sems):
        sw_s = sw_h

        def pipe(start, nk):
            pltpu.sync_copy(i0_h.at[pl.ds(start * E, nk * E)],
                            i0_v.at[pl.ds(0, nk * E)])
            pltpu.sync_copy(i1_h.at[pl.ds(start * E, nk * E)],
                            i1_v.at[pl.ds(0, nk * E)])

            def issue(i, p):
                sl = pl.ds(i * E, E)
                pltpu.async_copy(sw_s.at[i0_v.at[sl]], g0_v.at[p], sem0)
                pltpu.async_copy(sw_s.at[i1_v.at[sl]], g1_v.at[p], sem1)

            def half(j, p):
                i = 2 * j + p
                rows = pl.ds((start + i) * E, E)

                @pl.when(j > 0)
                def _():
                    _wait_bytes(sw_h.at[pl.ds(0, E), :], oc_v.at[p], sems)

                _wait_bytes(sw_h.at[pl.ds(0, E), :], g0_v.at[p], sem0)
                _wait_bytes(sw_h.at[pl.ds(0, E), :], g1_v.at[p], sem1)

                def row(e, _):
                    for v in range(8):
                        sl = pl.ds(v * 16, 16)
                        oc_v[p, e, sl] = g0_v[p, e, sl] + g1_v[p, e, sl]
                    return 0

                lax.fori_loop(0, E, row, 0)
                pltpu.async_copy(oc_v.at[p], out_h.at[rows, :], sems)

                @pl.when(i + 2 < nk)
                def _():
                    issue(i + 2, p)

            issue(0, 0)
            issue(1, 1)

            def body(j, _):
                half(j, 0)
                half(j, 1)
                return 0

            lax.fori_loop(0, nk // 2, body, 0)
            _wait_bytes(sw_h.at[pl.ds(0, E), :], oc_v.at[0], sems)
            _wait_bytes(sw_h.at[pl.ds(0, E), :], oc_v.at[1], sems)

        c_ax = lax.axis_index("c")
        s_ax = lax.axis_index("s")

        @pl.when(c_ax == 0)
        def _():
            pipe(s_ax * K0, K0)

        @pl.when(c_ax == 1)
        def _():
            pipe(16 * K0 + s_ax * K1, K1)

    return k(sw, i0, i1)


def _bond_fuse(bf, side, b, wt_next, need_bf, R=4096):
    out_shapes = [jax.ShapeDtypeStruct((NBP, DIM), jnp.float32)]
    out_specs = [pl.BlockSpec((R, DIM), lambda i: (i, 0))]
    if need_bf:
        out_shapes.append(jax.ShapeDtypeStruct((NBP, DIM), jnp.float32))
        out_specs.append(pl.BlockSpec((R, DIM), lambda i: (i, 0)))

    def body(bf_ref, sd_ref, b_ref, w_ref, o_ref, *rest):
        s = _sigmoid(bf_ref[...] + sd_ref[...] + b_ref[...])
        o_ref[...] = jnp.dot(s, w_ref[...], preferred_element_type=jnp.float32)
        if need_bf:
            rest[0][...] = s

    res = pl.pallas_call(
        body,
        grid=(NBP // R,),
        in_specs=[
            pl.BlockSpec((R, DIM), lambda i: (i, 0)),
            pl.BlockSpec((R, DIM), lambda i: (i, 0)),
            pl.BlockSpec((1, DIM), lambda i: (0, 0)),
            pl.BlockSpec((DIM, DIM), lambda i: (0, 0)),
        ],
        out_specs=out_specs,
        out_shape=out_shapes,
    )(bf, side, b.reshape(1, DIM), wt_next)
    return res if need_bf else (res[0], None)


def kernel(fingerprints, atom_degree_list, bond_feature, bond_degree_list,
           i_bond_j, adjacency, words, embed_table, W_bond, b_bond, W_nfc,
           b_nfc, W_sfc, b_sfc, W_sub, b_sub, W_fc, b_fc, W_out, b_out,
           W_int, b_int):
    def _padi(a, total, mod):
        pad = jnp.arange(total - a.shape[0], dtype=jnp.int32) % mod
        return jnp.concatenate([a.astype(jnp.int32), pad])

    fp = _padi(fingerprints, NPAD, 100000)
    xs = _sc_embed(embed_table, fp)[:N]

    for i in range(2):
        hs = _linear(xs, W_sub[i].T, b_sub[i], "relu", R=1000)
        xs = _adj_step(adjacency, hs, xs)

    bf, bw = _bf_init(bond_feature, W_bond.T, b_bond, W_nfc[0, :, DIM:].T)
    af_p = jnp.pad(xs, ((0, NPAD - N), (0, 0)))
    adlf = _padi(atom_degree_list.reshape(-1), NPAD * DEG, N)
    bdlf = _padi(bond_degree_list.reshape(-1), NPAD * DEG, NB)
    i0 = _padi(i_bond_j[:, 0], NBP, N)
    i1 = _padi(i_bond_j[:, 1], NBP, N)

    for i in range(3):
        aw = _linear(af_p, W_nfc[i, :, :DIM].T, None, None, R=1024)
        af_p = _sc_nf(aw, bw, adlf, bdlf, af_p, b_nfc[i])
        if i < 2:
            sw = _linear(af_p, W_sfc[i].T, None, None, R=1024)
            side = _sc_side(sw, i0, i1)
            bw, bf_new = _bond_fuse(bf, side, b_sfc[i],
                                    W_nfc[i + 1, :, DIM:].T, need_bf=(i == 0))
            if i == 0:
                bf = bf_new

    return _epilogue(xs, af_p[:N], words, W_fc.T, b_fc,
                     jnp.transpose(W_out, (0, 2, 1)), b_out, W_int.T, b_int)


# speedup vs baseline: 1.0905x; 1.0032x over previous
import functools

import jax
import jax.numpy as jnp
from jax import lax
from jax.experimental import pallas as pl
from jax.experimental.pallas import tpu as pltpu
from jax.experimental.pallas import tpu_sc as plsc

N = 10000
NPAD = 10240
DEG = 16
NB = 160000
NBP = 163840
DIM = 128
NW = 32


def _mesh():
    return plsc.VectorSubcoreMesh(
        core_axis_name="c", subcore_axis_name="s", num_cores=2, num_subcores=16
    )


def _wid():
    return lax.axis_index("s") * 2 + lax.axis_index("c")


def _linear(x, wt, b=None, act=None, R=640):
    M, K = x.shape
    Do = wt.shape[1]
    in_specs = [
        pl.BlockSpec((R, K), lambda i: (i, 0)),
        pl.BlockSpec((K, Do), lambda i: (0, 0)),
    ]
    args = [x, wt]
    if b is not None:
        in_specs.append(pl.BlockSpec((1, Do), lambda i: (0, 0)))
        args.append(b.reshape(1, Do))

    def body(*refs):
        x_ref, w_ref = refs[0], refs[1]
        o_ref = refs[-1]
        y = jnp.dot(x_ref[...], w_ref[...], preferred_element_type=jnp.float32)
        if b is not None:
            y = y + refs[2][...]
        if act == "relu":
            y = jnp.maximum(y, 0.0)
        o_ref[...] = y

    return pl.pallas_call(
        body,
        grid=(M // R,),
        in_specs=in_specs,
        out_specs=pl.BlockSpec((R, Do), lambda i: (i, 0)),
        out_shape=jax.ShapeDtypeStruct((M, Do), jnp.float32),
    )(*args)


def _adj_step(adjacency, hs, xs, BM=400):
    def body(a_ref, h_ref, x_ref, o_ref):
        o_ref[...] = x_ref[...] + jnp.dot(
            a_ref[...], h_ref[...], preferred_element_type=jnp.float32
        )

    return pl.pallas_call(
        body,
        grid=(N // BM,),
        in_specs=[
            pl.BlockSpec((BM, N), lambda i: (i, 0)),
            pl.BlockSpec((N, DIM), lambda i: (0, 0)),
            pl.BlockSpec((BM, DIM), lambda i: (i, 0)),
        ],
        out_specs=pl.BlockSpec((BM, DIM), lambda i: (i, 0)),
        out_shape=jax.ShapeDtypeStruct((N, DIM), jnp.float32),
    )(adjacency, hs, xs)


def _bf_init(bond_feature, wbond_t, b_bond, wb0_t):
    R = 1280
    nreal = NB // R

    def body(x_ref, w1_ref, b_ref, w2_ref, o1_ref, o2_ref):
        t = jnp.dot(x_ref[...], w1_ref[...],
                    preferred_element_type=jnp.float32) + b_ref[...]
        o1_ref[...] = t
        o2_ref[...] = jnp.dot(t, w2_ref[...],
                              preferred_element_type=jnp.float32)

    return pl.pallas_call(
        body,
        grid=(NBP // R,),
        in_specs=[
            pl.BlockSpec((R, 10), lambda i: (jnp.minimum(i, nreal - 1), 0)),
            pl.BlockSpec((10, DIM), lambda i: (0, 0)),
            pl.BlockSpec((1, DIM), lambda i: (0, 0)),
            pl.BlockSpec((DIM, DIM), lambda i: (0, 0)),
        ],
        out_specs=[
            pl.BlockSpec((R, DIM), lambda i: (i, 0)),
            pl.BlockSpec((R, DIM), lambda i: (i, 0)),
        ],
        out_shape=[
            jax.ShapeDtypeStruct((NBP, DIM), jnp.float32),
            jax.ShapeDtypeStruct((NBP, DIM), jnp.float32),
        ],
    )(bond_feature, wbond_t, b_bond.reshape(1, DIM), wb0_t)


def _epilogue(xs, af, words, wfc_t, bfc, wout_t, bout, wint_t, bint):
    def body(xs_ref, af_ref, w_ref, wfc_ref, bfc_ref, wout_ref, bout_ref,
             wint_ref, bint_ref, o_ref):
        s = jnp.sum(xs_ref[...] + af_ref[...], axis=0, keepdims=True) * (1.0 / N)
        m = jnp.dot(w_ref[...], wfc_ref[...], preferred_element_type=jnp.float32)
        m = m + bfc_ref[...]
        cat = jnp.concatenate([s, m], axis=1)
        for j in range(2):
            cat = jnp.dot(cat, wout_ref[j], preferred_element_type=jnp.float32)
            cat = jnp.maximum(cat + bout_ref[j], 0.0)
        o_ref[...] = (
            jnp.dot(cat, wint_ref[...], preferred_element_type=jnp.float32)
            + bint_ref[...]
        )

    return pl.pallas_call(
        body,
        out_shape=jax.ShapeDtypeStruct((1, 2), jnp.float32),
    )(xs, af, words.reshape(1, -1), wfc_t, bfc.reshape(1, -1), wout_t,
      bout.reshape(2, 1, 2 * DIM), wint_t, bint.reshape(1, -1))


def _sc_embed(table, idx):
    B = idx.shape[0]
    bpw = B // NW

    @functools.partial(
        pl.kernel,
        mesh=_mesh(),
        out_type=jax.ShapeDtypeStruct((B, DIM), jnp.float32),
        scratch_types=[
            pltpu.VMEM((bpw,), jnp.int32),
            pltpu.VMEM((bpw, DIM), jnp.float32),
            pltpu.SemaphoreType.DMA,
        ],
    )
    def k(table_h, idx_h, out_h, idx_v, rows_v, sem):
        base = _wid() * bpw
        pltpu.sync_copy(idx_h.at[pl.ds(base, bpw)], idx_v)
        cps = []
        for c in range(bpw // 64):
            sl = pl.ds(c * 64, 64)
            cps.append(pltpu.async_copy(table_h.at[idx_v.at[sl]],
                                        rows_v.at[sl, :], sem))
        for cp in cps:
            cp.wait()
        pltpu.sync_copy(rows_v, out_h.at[pl.ds(base, bpw), :])

    return k(table, idx)


def _sigmoid(x):
    return 1.0 / (1.0 + jnp.exp(-x))


K0 = 40
K1 = 40


def _wait_bytes(src_h, dst_ref, sem):
    pltpu.make_async_copy(src_h, dst_ref, sem).wait()


def _sc_nf(aw, bw, adl_flat, bdl_flat, af, bias):
    CA = 8
    R = CA * DEG
    KM = max(K0, K1)

    @functools.partial(
        pl.kernel,
        mesh=_mesh(),
        out_type=jax.ShapeDtypeStruct((NPAD, DIM), jnp.float32),
        scratch_types=[
            pltpu.VMEM((KM * R,), jnp.int32),
            pltpu.VMEM((KM * R,), jnp.int32),
            pltpu.VMEM((2, R, DIM), jnp.float32),
            pltpu.VMEM((2, R, DIM), jnp.float32),
            pltpu.VMEM((2, CA, DIM), jnp.float32),
            pltpu.VMEM((2, CA, DIM), jnp.float32),
            pltpu.VMEM((DIM,), jnp.float32),
            pltpu.SemaphoreType.DMA,
            pltpu.SemaphoreType.DMA,
            pltpu.SemaphoreType.DMA,
            pltpu.SemaphoreType.DMA,
        ],
    )
    def k(aw_h, bw_h, adl_h, bdl_h, af_h, bias_h, out_h,
          ia_v, ib_v, ar_v, br_v, af_v, oc_v, b_v, sema, semb, semf, sems):
        pltpu.sync_copy(bias_h, b_v)
        bias_vecs = [b_v[pl.ds(v * 16, 16)] for v in range(8)]

        def pipe(start, nk):
            pltpu.sync_copy(adl_h.at[pl.ds(start * R, nk * R)],
                            ia_v.at[pl.ds(0, nk * R)])
            pltpu.sync_copy(bdl_h.at[pl.ds(start * R, nk * R)],
                            ib_v.at[pl.ds(0, nk * R)])

            def issue(i, p):
                sl = pl.ds(i * R, R)
                rows = pl.ds((start + i) * CA, CA)
                pltpu.async_copy(aw_h.at[ia_v.at[sl]], ar_v.at[p], sema)
                pltpu.async_copy(bw_h.at[ib_v.at[sl]], br_v.at[p], semb)
                pltpu.async_copy(af_h.at[rows, :], af_v.at[p], semf)

            def half(j, p):
                i = 2 * j + p
                rows = pl.ds((start + i) * CA, CA)

                @pl.when(j > 0)
                def _():
                    _wait_bytes(af_h.at[rows, :], oc_v.at[p], sems)

                _wait_bytes(af_h.at[pl.ds(0, R), :], ar_v.at[p], sema)
                _wait_bytes(af_h.at[pl.ds(0, R), :], br_v.at[p], semb)
                _wait_bytes(af_h.at[pl.ds(0, CA), :], af_v.at[p], semf)

                def atom(a, _):
                    r0 = a * DEG

                    def dbody(d, accs):
                        r = r0 + d
                        out = []
                        for v in range(8):
                            sl = pl.ds(v * 16, 16)
                            x = ar_v[p, r, sl] + br_v[p, r, sl] + bias_vecs[v]
                            out.append(accs[v] + jnp.maximum(x, 0.0)
                                       + 0.01 * jnp.minimum(x, 0.0))
                        return tuple(out)

                    accs = lax.fori_loop(
                        0, DEG, dbody,
                        tuple(jnp.zeros((16,), jnp.float32) for _ in range(8)))
                    for v in range(8):
                        sl = pl.ds(v * 16, 16)
                        oc_v[p, a, sl] = _sigmoid(af_v[p, a, sl] + accs[v])
                    return 0

                lax.fori_loop(0, CA, atom, 0)
                pltpu.async_copy(oc_v.at[p], out_h.at[rows, :], sems)

                @pl.when(i + 2 < nk)
                def _():
                    issue(i + 2, p)

            issue(0, 0)
            issue(1, 1)

            def body(j, _):
                half(j, 0)
                half(j, 1)
                return 0

            lax.fori_loop(0, nk // 2, body, 0)
            _wait_bytes(af_h.at[pl.ds(0, CA), :], oc_v.at[0], sems)
            _wait_bytes(af_h.at[pl.ds(0, CA), :], oc_v.at[1], sems)

        c_ax = lax.axis_index("c")
        s_ax = lax.axis_index("s")

        @pl.when(c_ax == 0)
        def _():
            pipe(s_ax * K0, K0)

        @pl.when(c_ax == 1)
        def _():
            pipe(16 * K0 + s_ax * K1, K1)

    return k(aw, bw, adl_flat, bdl_flat, af, bias)


def _sc_side(sw, i0, i1):
    E = 128
    KM = max(K0, K1)

    @functools.partial(
        pl.kernel,
        mesh=_mesh(),
        out_type=jax.ShapeDtypeStruct((NBP, DIM), jnp.float32),
        scratch_types=[
            pltpu.VMEM((KM * E,), jnp.int32),
            pltpu.VMEM((KM * E,), jnp.int32),
            pltpu.VMEM((2, E, DIM), jnp.float32),
            pltpu.VMEM((2, E, DIM), jnp.float32),
            pltpu.VMEM((2, E, DIM), jnp.float32),
            pltpu.SemaphoreType.DMA,
            pltpu.SemaphoreType.DMA,
            pltpu.SemaphoreType.DMA,
        ],
    )
    def k(sw_h, i0_h, i1_h, out_h, i0_v, i1_v, g0_v, g1_v, oc_v,
          sem0, sem1, sems):
        sw_s = sw_h

        def pipe(start, nk):
            pltpu.sync_copy(i0_h.at[pl.ds(start * E, nk * E)],
                            i0_v.at[pl.ds(0, nk * E)])
            pltpu.sync_copy(i1_h.at[pl.ds(start * E, nk * E)],
                            i1_v.at[pl.ds(0, nk * E)])

            def issue(i, p):
                sl = pl.ds(i * E, E)
                pltpu.async_copy(sw_s.at[i0_v.at[sl]], g0_v.at[p], sem0)
                pltpu.async_copy(sw_s.at[i1_v.at[sl]], g1_v.at[p], sem1)

            def half(j, p):
                i = 2 * j + p
                rows = pl.ds((start + i) * E, E)

                @pl.when(j > 0)
                def _():
                    _wait_bytes(sw_h.at[pl.ds(0, E), :], oc_v.at[p], sems)

                _wait_bytes(sw_h.at[pl.ds(0, E), :], g0_v.at[p], sem0)
                _wait_bytes(sw_h.at[pl.ds(0, E), :], g1_v.at[p], sem1)

                def row(e, _):
                    for v in range(8):
                        sl = pl.ds(v * 16, 16)
                        oc_v[p, e, sl] = g0_v[p, e, sl] + g1_v[p, e, sl]
                    return 0

                lax.fori_loop(0, E, row, 0)
                pltpu.async_copy(oc_v.at[p], out_h.at[rows, :], sems)

                @pl.when(i + 2 < nk)
                def _():
                    issue(i + 2, p)

            issue(0, 0)
            issue(1, 1)

            def body(j, _):
                half(j, 0)
                half(j, 1)
                return 0

            lax.fori_loop(0, nk // 2, body, 0)
            _wait_bytes(sw_h.at[pl.ds(0, E), :], oc_v.at[0], sems)
            _wait_bytes(sw_h.at[pl.ds(0, E), :], oc_v.at[1], sems)

        c_ax = lax.axis_index("c")
        s_ax = lax.axis_index("s")

        @pl.when(c_ax == 0)
        def _():
            pipe(s_ax * K0, K0)

        @pl.when(c_ax == 1)
        def _():
            pipe(16 * K0 + s_ax * K1, K1)

    return k(sw, i0, i1)


def _bond_fuse(bf, side, b, wt_next, need_bf, R=8192):
    out_shapes = [jax.ShapeDtypeStruct((NBP, DIM), jnp.float32)]
    out_specs = [pl.BlockSpec((R, DIM), lambda i: (i, 0))]
    if need_bf:
        out_shapes.append(jax.ShapeDtypeStruct((NBP, DIM), jnp.float32))
        out_specs.append(pl.BlockSpec((R, DIM), lambda i: (i, 0)))

    def body(bf_ref, sd_ref, b_ref, w_ref, o_ref, *rest):
        s = _sigmoid(bf_ref[...] + sd_ref[...] + b_ref[...])
        o_ref[...] = jnp.dot(s, w_ref[...], preferred_element_type=jnp.float32)
        if need_bf:
            rest[0][...] = s

    res = pl.pallas_call(
        body,
        grid=(NBP // R,),
        in_specs=[
            pl.BlockSpec((R, DIM), lambda i: (i, 0)),
            pl.BlockSpec((R, DIM), lambda i: (i, 0)),
            pl.BlockSpec((1, DIM), lambda i: (0, 0)),
            pl.BlockSpec((DIM, DIM), lambda i: (0, 0)),
        ],
        out_specs=out_specs,
        out_shape=out_shapes,
    )(bf, side, b.reshape(1, DIM), wt_next)
    return res if need_bf else (res[0], None)


def kernel(fingerprints, atom_degree_list, bond_feature, bond_degree_list,
           i_bond_j, adjacency, words, embed_table, W_bond, b_bond, W_nfc,
           b_nfc, W_sfc, b_sfc, W_sub, b_sub, W_fc, b_fc, W_out, b_out,
           W_int, b_int):
    def _padi(a, total, mod):
        pad = jnp.arange(total - a.shape[0], dtype=jnp.int32) % mod
        return jnp.concatenate([a.astype(jnp.int32), pad])

    fp = _padi(fingerprints, NPAD, 100000)
    xs = _sc_embed(embed_table, fp)[:N]

    for i in range(2):
        hs = _linear(xs, W_sub[i].T, b_sub[i], "relu", R=1000)
        xs = _adj_step(adjacency, hs, xs)

    bf, bw = _bf_init(bond_feature, W_bond.T, b_bond, W_nfc[0, :, DIM:].T)
    af_p = jnp.pad(xs, ((0, NPAD - N), (0, 0)))
    adlf = _padi(atom_degree_list.reshape(-1), NPAD * DEG, N)
    bdlf = _padi(bond_degree_list.reshape(-1), NPAD * DEG, NB)
    i0 = _padi(i_bond_j[:, 0], NBP, N)
    i1 = _padi(i_bond_j[:, 1], NBP, N)

    for i in range(3):
        aw = _linear(af_p, W_nfc[i, :, :DIM].T, None, None, R=1024)
        af_p = _sc_nf(aw, bw, adlf, bdlf, af_p, b_nfc[i])
        if i < 2:
            sw = _linear(af_p, W_sfc[i].T, None, None, R=1024)
            side = _sc_side(sw, i0, i1)
            bw, bf_new = _bond_fuse(bf, side, b_sfc[i],
                                    W_nfc[i + 1, :, DIM:].T, need_bf=(i == 0))
            if i == 0:
                bf = bf_new

    return _epilogue(xs, af_p[:N], words, W_fc.T, b_fc,
                     jnp.transpose(W_out, (0, 2, 1)), b_out, W_int.T, b_int)
